# baseline (speedup 1.0000x reference)
import functools

import jax
import jax.numpy as jnp
from jax import lax
from jax.experimental import pallas as pl
from jax.experimental.pallas import tpu as pltpu
from jax.experimental.pallas import tpu_sc as plsc

_EPS = 1e-08
_M = 1024
_RPC = 128
_SPLIT = 1


def _stats1_kernel(x_ref, w_ref, b_ref, acc_ref):
    i = pl.program_id(0)
    y = jnp.dot(x_ref[...], w_ref[...].T, preferred_element_type=jnp.float32)
    y = y + b_ref[...]
    s = jnp.sum(y, axis=0, keepdims=True)
    ss = jnp.sum(y * y, axis=0, keepdims=True)
    blk = jnp.concatenate([s, ss], axis=0)

    @pl.when(i == 0)
    def _():
        acc_ref[...] = blk

    @pl.when(i > 0)
    def _():
        acc_ref[...] += blk


def _f2_kernel(x_ref, w_ref, b_ref, g_ref, beta_ref, out_ref):
    y = jnp.dot(x_ref[...], w_ref[...].T, preferred_element_type=jnp.float32)
    y = y + b_ref[...]
    m = jnp.mean(y, axis=0, keepdims=True)
    v = jnp.mean((y - m) ** 2, axis=0, keepdims=True)
    f2 = jnp.maximum(
        (y - m) / jnp.sqrt(v + 1e-5) * g_ref[...] + beta_ref[...], 0.0
    )
    out_ref[...] = jnp.concatenate([f2, jnp.zeros_like(f2)], axis=1)


def _f1_and_top3(x1_ref, p1_ref, p2t_ref, w1_ref, b1_ref, g1_ref, beta1_ref,
                 stats_ref, n_rows):
    mean = stats_ref[0:1, :] / n_rows
    var = stats_ref[1:2, :] / n_rows - mean * mean
    x1 = x1_ref[0]
    y1 = jnp.dot(x1, w1_ref[...].T, preferred_element_type=jnp.float32)
    y1 = y1 + b1_ref[...]
    f1 = jnp.maximum(
        (y1 - mean) / jnp.sqrt(var + 1e-5) * g1_ref[...] + beta1_ref[...], 0.0
    )

    p1 = p1_ref[0]
    p2t = p2t_ref[0]
    q2 = jnp.sum(p1 * p1, axis=1, keepdims=True)
    r2 = jnp.sum(p2t * p2t, axis=0, keepdims=True)
    cross = jnp.dot(p1, p2t, preferred_element_type=jnp.float32)
    d2 = jnp.maximum(q2 + r2 - 2.0 * cross, 0.0)

    iota = jax.lax.broadcasted_iota(jnp.int32, d2.shape, 1)
    hi = jnp.int32(-2048)
    ukey = (jax.lax.bitcast_convert_type(d2, jnp.int32) & hi) | iota
    fkey = jax.lax.bitcast_convert_type(ukey + jnp.int32(0x00800000),
                                        jnp.float32)
    fbig = jnp.float32(3e38)
    m1 = jnp.min(fkey, axis=1, keepdims=True)
    u2 = jnp.where(fkey == m1, fbig, fkey)
    m2 = jnp.min(u2, axis=1, keepdims=True)
    m3 = jnp.min(jnp.where(u2 == m2, fbig, u2), axis=1, keepdims=True)
    ikeys = [jax.lax.bitcast_convert_type(mk, jnp.int32)
             for mk in (m1, m2, m3)]
    mins = [
        jax.lax.bitcast_convert_type(
            (ik - jnp.int32(0x00800000)) & hi, jnp.float32)
        for ik in ikeys
    ]
    recips = [1.0 / (jnp.sqrt(mn) + _EPS) for mn in mins]
    norm = recips[0] + recips[1] + recips[2]
    weights = [rc / norm for rc in recips]
    return f1, fkey, (m1, m2, m3), ikeys, weights


def _knn_emit_kernel(x1_ref, p1_ref, p2t_ref, w1_ref, b1_ref, g1_ref,
                     beta1_ref, stats_ref, f1_ref, w3_ref, idx3_ref,
                     *, n_rows, n2):
    f1, _, _, ikeys, weights = _f1_and_top3(
        x1_ref, p1_ref, p2t_ref, w1_ref, b1_ref, g1_ref, beta1_ref,
        stats_ref, n_rows)
    f1_ref[0] = f1
    cols = [ik & jnp.int32(2047) for ik in ikeys]
    gbase = pl.program_id(0) * n2
    w3_ref[0] = jnp.concatenate(weights, axis=1)
    idx3_ref[0] = jnp.concatenate(cols, axis=1) + gbase


def _knn_onehot_kernel(x1_ref, p1_ref, p2t_ref, f2_ref, w1_ref, b1_ref,
                       g1_ref, beta1_ref, stats_ref, out_ref,
                       *, n_rows, n2, feat):
    f1, fkey, (m1, m2, m3), _, weights = _f1_and_top3(
        x1_ref, p1_ref, p2t_ref, w1_ref, b1_ref, g1_ref, beta1_ref,
        stats_ref, n_rows)
    oh = jnp.where(
        fkey == m1, weights[0],
        jnp.where(fkey == m2, weights[1],
                  jnp.where(fkey == m3, weights[2], 0.0)),
    )
    f2b = f2_ref[0][:, :feat]
    new = jnp.dot(oh, f2b, preferred_element_type=jnp.float32)
    out_ref[0] = f1 + new


def _make_sc_combine(n_rows, feat):
    info = plsc.get_sparse_core_info()
    nw = info.num_cores * info.num_subcores
    rows_per_w = n_rows // nw
    nchunks = rows_per_w // _RPC
    ni = _RPC * 3
    mesh = plsc.VectorSubcoreMesh(core_axis_name="c", subcore_axis_name="s")

    @functools.partial(
        pl.kernel, mesh=mesh,
        out_type=jax.ShapeDtypeStruct((n_rows, feat), jnp.float32),
        scratch_types=[
            pltpu.VMEM((ni,), jnp.int32),
            pltpu.VMEM((ni + 16,), jnp.float32),
            pltpu.VMEM((ni, 2 * feat), jnp.float32),
            pltpu.VMEM((_RPC, feat), jnp.float32),
            pltpu.SemaphoreType.DMA,
        ],
    )
    def sc_combine(f2_hbm, idx_hbm, w_hbm, f1_hbm, out_hbm,
                   idx_v, w_v, rows_v, f1_v, sem):
        wid = lax.axis_index("s") * info.num_cores + lax.axis_index("c")
        base = wid * rows_per_w

        def chunk_body(g, carry):
            row0 = base + g * _RPC
            pltpu.sync_copy(idx_hbm.at[pl.ds(row0 * 3, ni)], idx_v)
            pltpu.sync_copy(w_hbm.at[pl.ds(row0 * 3, ni)],
                            w_v.at[pl.ds(0, ni)])
            pltpu.sync_copy(f1_hbm.at[pl.ds(row0, _RPC)], f1_v)
            copies = [
                pltpu.async_copy(
                    f2_hbm.at[idx_v.at[pl.ds(q * 128, 128)]],
                    rows_v.at[pl.ds(q * 128, 128)],
                    sem,
                )
                for q in range(ni // 128)
            ]
            for c in copies:
                c.wait()

            def row_body(r, c2):
                wvec = w_v[pl.ds(r * 3, 16)]
                for j in range(feat // 16):
                    sl = pl.ds(j * 16, 16)
                    acc = f1_v[r, sl]
                    for k in range(3):
                        acc = acc + wvec[k] * rows_v[r * 3 + k, sl]
                    f1_v[r, sl] = acc
                return c2

            lax.fori_loop(0, _RPC, row_body, 0, unroll=2)
            pltpu.sync_copy(f1_v, out_hbm.at[pl.ds(row0, _RPC)])
            return carry

        lax.fori_loop(0, nchunks, chunk_body, 0)

    return sc_combine


def kernel(feats1, points1, feats2, points2, W1, b1, g1, beta1, W2, b2, g2,
           beta2):
    B, N1, C1 = feats1.shape
    _, N2, C2 = feats2.shape
    C = W1.shape[0]
    x1 = feats1.reshape(B * N1, C1)
    x2 = feats2.reshape(B * N2, C2)
    b1r = b1.reshape(1, C)
    g1r = g1.reshape(1, C)
    beta1r = beta1.reshape(1, C)
    b2r = b2.reshape(1, C)
    g2r = g2.reshape(1, C)
    beta2r = beta2.reshape(1, C)

    chunk = 4096
    nchunks = (B * N1) // chunk
    stats = pl.pallas_call(
        _stats1_kernel,
        grid=(nchunks,),
        in_specs=[
            pl.BlockSpec((chunk, C1), lambda i: (i, 0)),
            pl.BlockSpec((C, C1), lambda i: (0, 0)),
            pl.BlockSpec((1, C), lambda i: (0, 0)),
        ],
        out_specs=pl.BlockSpec((2, C), lambda i: (0, 0)),
        out_shape=jax.ShapeDtypeStruct((2, C), jnp.float32),
    )(x1, W1, b1r)

    f2 = pl.pallas_call(
        _f2_kernel,
        out_shape=jax.ShapeDtypeStruct((B * N2, 2 * C), jnp.float32),
    )(x2, W2, b2r, g2r, beta2r)

    p2t = points2.transpose(0, 2, 1)

    nba = _SPLIT
    nbb = N1 // _M - nba
    n1a = nba * _M

    f1a, w3, idx3 = pl.pallas_call(
        functools.partial(_knn_emit_kernel, n_rows=float(B * N1), n2=N2),
        grid=(B, nba),
        in_specs=[
            pl.BlockSpec((1, _M, C1), lambda b, n: (b, n, 0)),
            pl.BlockSpec((1, _M, 3), lambda b, n: (b, n, 0)),
            pl.BlockSpec((1, 3, N2), lambda b, n: (b, 0, 0)),
            pl.BlockSpec((C, C1), lambda b, n: (0, 0)),
            pl.BlockSpec((1, C), lambda b, n: (0, 0)),
            pl.BlockSpec((1, C), lambda b, n: (0, 0)),
            pl.BlockSpec((1, C), lambda b, n: (0, 0)),
            pl.BlockSpec((2, C), lambda b, n: (0, 0)),
        ],
        out_specs=(
            pl.BlockSpec((1, _M, C), lambda b, n: (b, n, 0)),
            pl.BlockSpec((1, _M, 3), lambda b, n: (b, n, 0)),
            pl.BlockSpec((1, _M, 3), lambda b, n: (b, n, 0)),
        ),
        out_shape=(
            jax.ShapeDtypeStruct((B, n1a, C), jnp.float32),
            jax.ShapeDtypeStruct((B, n1a, 3), jnp.float32),
            jax.ShapeDtypeStruct((B, n1a, 3), jnp.int32),
        ),
    )(feats1, points1, p2t, W1, b1r, g1r, beta1r, stats)

    sc_combine = _make_sc_combine(B * n1a, C)
    out_a = sc_combine(
        f2,
        idx3.reshape(B * n1a * 3),
        w3.reshape(B * n1a * 3),
        f1a.reshape(B * n1a, C),
    )

    out_b = pl.pallas_call(
        functools.partial(_knn_onehot_kernel, n_rows=float(B * N1), n2=N2,
                          feat=C),
        grid=(B, nbb),
        in_specs=[
            pl.BlockSpec((1, _M, C1), lambda b, n: (b, n + nba, 0)),
            pl.BlockSpec((1, _M, 3), lambda b, n: (b, n + nba, 0)),
            pl.BlockSpec((1, 3, N2), lambda b, n: (b, 0, 0)),
            pl.BlockSpec((1, N2, 2 * C), lambda b, n: (b, 0, 0)),
            pl.BlockSpec((C, C1), lambda b, n: (0, 0)),
            pl.BlockSpec((1, C), lambda b, n: (0, 0)),
            pl.BlockSpec((1, C), lambda b, n: (0, 0)),
            pl.BlockSpec((1, C), lambda b, n: (0, 0)),
            pl.BlockSpec((2, C), lambda b, n: (0, 0)),
        ],
        out_specs=pl.BlockSpec((1, _M, C), lambda b, n: (b, n, 0)),
        out_shape=jax.ShapeDtypeStruct((B, N1 - n1a, C), jnp.float32),
    )(feats1, points1, p2t, f2.reshape(B, N2, 2 * C), W1, b1r, g1r, beta1r,
      stats)

    out = jnp.concatenate([out_a.reshape(B, n1a, C), out_b], axis=1)
    return (out, points1)


# speedup vs baseline: 1.0254x; 1.0254x over previous
import functools

import jax
import jax.numpy as jnp
from jax import lax
from jax.experimental import pallas as pl
from jax.experimental.pallas import tpu as pltpu
from jax.experimental.pallas import tpu_sc as plsc

_EPS = 1e-08
_M = 1024
_RPC = 128
_SPLIT = 3


def _stats1_kernel(x_ref, w_ref, b_ref, acc_ref):
    i = pl.program_id(0)
    y = jnp.dot(x_ref[...], w_ref[...].T, preferred_element_type=jnp.float32)
    y = y + b_ref[...]
    s = jnp.sum(y, axis=0, keepdims=True)
    ss = jnp.sum(y * y, axis=0, keepdims=True)
    blk = jnp.concatenate([s, ss], axis=0)

    @pl.when(i == 0)
    def _():
        acc_ref[...] = blk

    @pl.when(i > 0)
    def _():
        acc_ref[...] += blk


def _f2_kernel(x_ref, w_ref, b_ref, g_ref, beta_ref, out_ref):
    y = jnp.dot(x_ref[...], w_ref[...].T, preferred_element_type=jnp.float32)
    y = y + b_ref[...]
    m = jnp.mean(y, axis=0, keepdims=True)
    v = jnp.mean((y - m) ** 2, axis=0, keepdims=True)
    f2 = jnp.maximum(
        (y - m) / jnp.sqrt(v + 1e-5) * g_ref[...] + beta_ref[...], 0.0
    )
    out_ref[...] = jnp.concatenate([f2, jnp.zeros_like(f2)], axis=1)


def _f1_and_top3(x1_ref, p1_ref, p2t_ref, w1_ref, b1_ref, g1_ref, beta1_ref,
                 stats_ref, n_rows):
    mean = stats_ref[0:1, :] / n_rows
    var = stats_ref[1:2, :] / n_rows - mean * mean
    x1 = x1_ref[0]
    y1 = jnp.dot(x1, w1_ref[...].T, preferred_element_type=jnp.float32)
    y1 = y1 + b1_ref[...]
    f1 = jnp.maximum(
        (y1 - mean) / jnp.sqrt(var + 1e-5) * g1_ref[...] + beta1_ref[...], 0.0
    )

    p1 = p1_ref[0]
    p2t = p2t_ref[0]
    q2 = jnp.sum(p1 * p1, axis=1, keepdims=True)
    r2 = jnp.sum(p2t * p2t, axis=0, keepdims=True)
    cross = jnp.dot(p1, p2t, preferred_element_type=jnp.float32)
    d2 = jnp.maximum(q2 + r2 - 2.0 * cross, 0.0)

    iota = jax.lax.broadcasted_iota(jnp.int32, d2.shape, 1)
    hi = jnp.int32(-2048)
    ukey = (jax.lax.bitcast_convert_type(d2, jnp.int32) & hi) | iota
    fkey = jax.lax.bitcast_convert_type(ukey + jnp.int32(0x00800000),
                                        jnp.float32)
    fbig = jnp.float32(3e38)
    m1 = jnp.min(fkey, axis=1, keepdims=True)
    u2 = jnp.where(fkey == m1, fbig, fkey)
    m2 = jnp.min(u2, axis=1, keepdims=True)
    m3 = jnp.min(jnp.where(u2 == m2, fbig, u2), axis=1, keepdims=True)
    ikeys = [jax.lax.bitcast_convert_type(mk, jnp.int32)
             for mk in (m1, m2, m3)]
    mins = [
        jax.lax.bitcast_convert_type(
            (ik - jnp.int32(0x00800000)) & hi, jnp.float32)
        for ik in ikeys
    ]
    recips = [1.0 / (jnp.sqrt(mn) + _EPS) for mn in mins]
    norm = recips[0] + recips[1] + recips[2]
    weights = [rc / norm for rc in recips]
    return f1, fkey, (m1, m2, m3), ikeys, weights


def _knn_emit_kernel(x1_ref, p1_ref, p2t_ref, w1_ref, b1_ref, g1_ref,
                     beta1_ref, stats_ref, f1_ref, w3_ref, idx3_ref,
                     *, n_rows, n2):
    f1, _, _, ikeys, weights = _f1_and_top3(
        x1_ref, p1_ref, p2t_ref, w1_ref, b1_ref, g1_ref, beta1_ref,
        stats_ref, n_rows)
    f1_ref[0] = f1
    cols = [ik & jnp.int32(2047) for ik in ikeys]
    gbase = pl.program_id(0) * n2
    w3_ref[0] = jnp.concatenate(weights, axis=1)
    idx3_ref[0] = jnp.concatenate(cols, axis=1) + gbase


def _knn_onehot_kernel(x1_ref, p1_ref, p2t_ref, f2_ref, w1_ref, b1_ref,
                       g1_ref, beta1_ref, stats_ref, out_ref,
                       *, n_rows, n2, feat):
    f1, fkey, (m1, m2, m3), _, weights = _f1_and_top3(
        x1_ref, p1_ref, p2t_ref, w1_ref, b1_ref, g1_ref, beta1_ref,
        stats_ref, n_rows)
    oh = jnp.where(
        fkey == m1, weights[0],
        jnp.where(fkey == m2, weights[1],
                  jnp.where(fkey == m3, weights[2], 0.0)),
    )
    f2b = f2_ref[0][:, :feat]
    new = jnp.dot(oh, f2b, preferred_element_type=jnp.float32)
    out_ref[0] = f1 + new


def _make_sc_combine(n_rows, feat):
    info = plsc.get_sparse_core_info()
    nw = info.num_cores * info.num_subcores
    rows_per_w = n_rows // nw
    nchunks = rows_per_w // _RPC
    ni = _RPC * 3
    mesh = plsc.VectorSubcoreMesh(core_axis_name="c", subcore_axis_name="s")

    @functools.partial(
        pl.kernel, mesh=mesh,
        out_type=jax.ShapeDtypeStruct((n_rows, feat), jnp.float32),
        scratch_types=[
            pltpu.VMEM((ni,), jnp.int32),
            pltpu.VMEM((ni + 16,), jnp.float32),
            pltpu.VMEM((ni, 2 * feat), jnp.float32),
            pltpu.VMEM((_RPC, feat), jnp.float32),
            pltpu.SemaphoreType.DMA,
        ],
    )
    def sc_combine(f2_hbm, idx_hbm, w_hbm, f1_hbm, out_hbm,
                   idx_v, w_v, rows_v, f1_v, sem):
        wid = lax.axis_index("s") * info.num_cores + lax.axis_index("c")
        base = wid * rows_per_w

        def chunk_body(g, carry):
            row0 = base + g * _RPC
            pltpu.sync_copy(idx_hbm.at[pl.ds(row0 * 3, ni)], idx_v)
            pltpu.sync_copy(w_hbm.at[pl.ds(row0 * 3, ni)],
                            w_v.at[pl.ds(0, ni)])
            pltpu.sync_copy(f1_hbm.at[pl.ds(row0, _RPC)], f1_v)
            copies = [
                pltpu.async_copy(
                    f2_hbm.at[idx_v.at[pl.ds(q * 128, 128)]],
                    rows_v.at[pl.ds(q * 128, 128)],
                    sem,
                )
                for q in range(ni // 128)
            ]
            for c in copies:
                c.wait()

            def row_body(r, c2):
                wvec = w_v[pl.ds(r * 3, 16)]
                for j in range(feat // 16):
                    sl = pl.ds(j * 16, 16)
                    acc = f1_v[r, sl]
                    for k in range(3):
                        acc = acc + wvec[k] * rows_v[r * 3 + k, sl]
                    f1_v[r, sl] = acc
                return c2

            lax.fori_loop(0, _RPC, row_body, 0, unroll=2)
            pltpu.sync_copy(f1_v, out_hbm.at[pl.ds(row0, _RPC)])
            return carry

        lax.fori_loop(0, nchunks, chunk_body, 0)

    return sc_combine


def kernel(feats1, points1, feats2, points2, W1, b1, g1, beta1, W2, b2, g2,
           beta2):
    B, N1, C1 = feats1.shape
    _, N2, C2 = feats2.shape
    C = W1.shape[0]
    x1 = feats1.reshape(B * N1, C1)
    x2 = feats2.reshape(B * N2, C2)
    b1r = b1.reshape(1, C)
    g1r = g1.reshape(1, C)
    beta1r = beta1.reshape(1, C)
    b2r = b2.reshape(1, C)
    g2r = g2.reshape(1, C)
    beta2r = beta2.reshape(1, C)

    chunk = 4096
    nchunks = (B * N1) // chunk
    stats = pl.pallas_call(
        _stats1_kernel,
        grid=(nchunks,),
        in_specs=[
            pl.BlockSpec((chunk, C1), lambda i: (i, 0)),
            pl.BlockSpec((C, C1), lambda i: (0, 0)),
            pl.BlockSpec((1, C), lambda i: (0, 0)),
        ],
        out_specs=pl.BlockSpec((2, C), lambda i: (0, 0)),
        out_shape=jax.ShapeDtypeStruct((2, C), jnp.float32),
    )(x1, W1, b1r)

    f2 = pl.pallas_call(
        _f2_kernel,
        out_shape=jax.ShapeDtypeStruct((B * N2, 2 * C), jnp.float32),
    )(x2, W2, b2r, g2r, beta2r)

    p2t = points2.transpose(0, 2, 1)

    nba = _SPLIT
    nbb = N1 // _M - nba
    n1a = nba * _M

    f1a, w3, idx3 = pl.pallas_call(
        functools.partial(_knn_emit_kernel, n_rows=float(B * N1), n2=N2),
        grid=(B, nba),
        in_specs=[
            pl.BlockSpec((1, _M, C1), lambda b, n: (b, n, 0)),
            pl.BlockSpec((1, _M, 3), lambda b, n: (b, n, 0)),
            pl.BlockSpec((1, 3, N2), lambda b, n: (b, 0, 0)),
            pl.BlockSpec((C, C1), lambda b, n: (0, 0)),
            pl.BlockSpec((1, C), lambda b, n: (0, 0)),
            pl.BlockSpec((1, C), lambda b, n: (0, 0)),
            pl.BlockSpec((1, C), lambda b, n: (0, 0)),
            pl.BlockSpec((2, C), lambda b, n: (0, 0)),
        ],
        out_specs=(
            pl.BlockSpec((1, _M, C), lambda b, n: (b, n, 0)),
            pl.BlockSpec((1, _M, 3), lambda b, n: (b, n, 0)),
            pl.BlockSpec((1, _M, 3), lambda b, n: (b, n, 0)),
        ),
        out_shape=(
            jax.ShapeDtypeStruct((B, n1a, C), jnp.float32),
            jax.ShapeDtypeStruct((B, n1a, 3), jnp.float32),
            jax.ShapeDtypeStruct((B, n1a, 3), jnp.int32),
        ),
    )(feats1, points1, p2t, W1, b1r, g1r, beta1r, stats)

    sc_combine = _make_sc_combine(B * n1a, C)
    out_a = sc_combine(
        f2,
        idx3.reshape(B * n1a * 3),
        w3.reshape(B * n1a * 3),
        f1a.reshape(B * n1a, C),
    )

    out_b = pl.pallas_call(
        functools.partial(_knn_onehot_kernel, n_rows=float(B * N1), n2=N2,
                          feat=C),
        grid=(B, nbb),
        in_specs=[
            pl.BlockSpec((1, _M, C1), lambda b, n: (b, n + nba, 0)),
            pl.BlockSpec((1, _M, 3), lambda b, n: (b, n + nba, 0)),
            pl.BlockSpec((1, 3, N2), lambda b, n: (b, 0, 0)),
            pl.BlockSpec((1, N2, 2 * C), lambda b, n: (b, 0, 0)),
            pl.BlockSpec((C, C1), lambda b, n: (0, 0)),
            pl.BlockSpec((1, C), lambda b, n: (0, 0)),
            pl.BlockSpec((1, C), lambda b, n: (0, 0)),
            pl.BlockSpec((1, C), lambda b, n: (0, 0)),
            pl.BlockSpec((2, C), lambda b, n: (0, 0)),
        ],
        out_specs=pl.BlockSpec((1, _M, C), lambda b, n: (b, n, 0)),
        out_shape=jax.ShapeDtypeStruct((B, N1 - n1a, C), jnp.float32),
    )(feats1, points1, p2t, f2.reshape(B, N2, 2 * C), W1, b1r, g1r, beta1r,
      stats)

    out = jnp.concatenate([out_a.reshape(B, n1a, C), out_b], axis=1)
    return (out, points1)


# speedup vs baseline: 1.0422x; 1.0164x over previous
import functools

import jax
import jax.numpy as jnp
from jax import lax
from jax.experimental import pallas as pl
from jax.experimental.pallas import tpu as pltpu
from jax.experimental.pallas import tpu_sc as plsc

_EPS = 1e-08
_M = 1024
_RPC = 128
_SPLIT = 4


def _stats1_kernel(x_ref, w_ref, b_ref, acc_ref):
    i = pl.program_id(0)
    y = jnp.dot(x_ref[...], w_ref[...].T, preferred_element_type=jnp.float32)
    y = y + b_ref[...]
    s = jnp.sum(y, axis=0, keepdims=True)
    ss = jnp.sum(y * y, axis=0, keepdims=True)
    blk = jnp.concatenate([s, ss], axis=0)

    @pl.when(i == 0)
    def _():
        acc_ref[...] = blk

    @pl.when(i > 0)
    def _():
        acc_ref[...] += blk


def _f2_kernel(x_ref, w_ref, b_ref, g_ref, beta_ref, out_ref):
    y = jnp.dot(x_ref[...], w_ref[...].T, preferred_element_type=jnp.float32)
    y = y + b_ref[...]
    m = jnp.mean(y, axis=0, keepdims=True)
    v = jnp.mean((y - m) ** 2, axis=0, keepdims=True)
    f2 = jnp.maximum(
        (y - m) / jnp.sqrt(v + 1e-5) * g_ref[...] + beta_ref[...], 0.0
    )
    out_ref[...] = jnp.concatenate([f2, jnp.zeros_like(f2)], axis=1)


def _f1_and_top3(x1_ref, p1_ref, p2t_ref, w1_ref, b1_ref, g1_ref, beta1_ref,
                 stats_ref, n_rows):
    mean = stats_ref[0:1, :] / n_rows
    var = stats_ref[1:2, :] / n_rows - mean * mean
    x1 = x1_ref[0]
    y1 = jnp.dot(x1, w1_ref[...].T, preferred_element_type=jnp.float32)
    y1 = y1 + b1_ref[...]
    f1 = jnp.maximum(
        (y1 - mean) / jnp.sqrt(var + 1e-5) * g1_ref[...] + beta1_ref[...], 0.0
    )

    p1 = p1_ref[0]
    p2t = p2t_ref[0]
    q2 = jnp.sum(p1 * p1, axis=1, keepdims=True)
    r2 = jnp.sum(p2t * p2t, axis=0, keepdims=True)
    cross = jnp.dot(p1, p2t, preferred_element_type=jnp.float32)
    d2 = jnp.maximum(q2 + r2 - 2.0 * cross, 0.0)

    iota = jax.lax.broadcasted_iota(jnp.int32, d2.shape, 1)
    hi = jnp.int32(-2048)
    ukey = (jax.lax.bitcast_convert_type(d2, jnp.int32) & hi) | iota
    fkey = jax.lax.bitcast_convert_type(ukey + jnp.int32(0x00800000),
                                        jnp.float32)
    fbig = jnp.float32(3e38)
    m1 = jnp.min(fkey, axis=1, keepdims=True)
    u2 = jnp.where(fkey == m1, fbig, fkey)
    m2 = jnp.min(u2, axis=1, keepdims=True)
    m3 = jnp.min(jnp.where(u2 == m2, fbig, u2), axis=1, keepdims=True)
    ikeys = [jax.lax.bitcast_convert_type(mk, jnp.int32)
             for mk in (m1, m2, m3)]
    mins = [
        jax.lax.bitcast_convert_type(
            (ik - jnp.int32(0x00800000)) & hi, jnp.float32)
        for ik in ikeys
    ]
    recips = [1.0 / (jnp.sqrt(mn) + _EPS) for mn in mins]
    norm = recips[0] + recips[1] + recips[2]
    weights = [rc / norm for rc in recips]
    return f1, fkey, (m1, m2, m3), ikeys, weights


def _knn_emit_kernel(x1_ref, p1_ref, p2t_ref, w1_ref, b1_ref, g1_ref,
                     beta1_ref, stats_ref, f1_ref, w3_ref, idx3_ref,
                     *, n_rows, n2):
    f1, _, _, ikeys, weights = _f1_and_top3(
        x1_ref, p1_ref, p2t_ref, w1_ref, b1_ref, g1_ref, beta1_ref,
        stats_ref, n_rows)
    f1_ref[0] = f1
    cols = [ik & jnp.int32(2047) for ik in ikeys]
    gbase = pl.program_id(0) * n2
    w3_ref[0] = jnp.concatenate(weights, axis=1)
    idx3_ref[0] = jnp.concatenate(cols, axis=1) + gbase


def _knn_onehot_kernel(x1_ref, p1_ref, p2t_ref, f2_ref, w1_ref, b1_ref,
                       g1_ref, beta1_ref, stats_ref, out_ref,
                       *, n_rows, n2, feat):
    f1, fkey, (m1, m2, m3), _, weights = _f1_and_top3(
        x1_ref, p1_ref, p2t_ref, w1_ref, b1_ref, g1_ref, beta1_ref,
        stats_ref, n_rows)
    oh = jnp.where(
        fkey == m1, weights[0],
        jnp.where(fkey == m2, weights[1],
                  jnp.where(fkey == m3, weights[2], 0.0)),
    )
    f2b = f2_ref[0][:, :feat]
    new = jnp.dot(oh, f2b, preferred_element_type=jnp.float32)
    out_ref[0] = f1 + new


def _make_sc_combine(n_rows, feat):
    info = plsc.get_sparse_core_info()
    nw = info.num_cores * info.num_subcores
    rows_per_w = n_rows // nw
    nchunks = rows_per_w // _RPC
    ni = _RPC * 3
    mesh = plsc.VectorSubcoreMesh(core_axis_name="c", subcore_axis_name="s")

    @functools.partial(
        pl.kernel, mesh=mesh,
        out_type=jax.ShapeDtypeStruct((n_rows, feat), jnp.float32),
        scratch_types=[
            pltpu.VMEM((ni,), jnp.int32),
            pltpu.VMEM((ni + 16,), jnp.float32),
            pltpu.VMEM((ni, 2 * feat), jnp.float32),
            pltpu.VMEM((_RPC, feat), jnp.float32),
            pltpu.SemaphoreType.DMA,
        ],
    )
    def sc_combine(f2_hbm, idx_hbm, w_hbm, f1_hbm, out_hbm,
                   idx_v, w_v, rows_v, f1_v, sem):
        wid = lax.axis_index("s") * info.num_cores + lax.axis_index("c")
        base = wid * rows_per_w

        def chunk_body(g, carry):
            row0 = base + g * _RPC
            pltpu.sync_copy(idx_hbm.at[pl.ds(row0 * 3, ni)], idx_v)
            pltpu.sync_copy(w_hbm.at[pl.ds(row0 * 3, ni)],
                            w_v.at[pl.ds(0, ni)])
            pltpu.sync_copy(f1_hbm.at[pl.ds(row0, _RPC)], f1_v)
            copies = [
                pltpu.async_copy(
                    f2_hbm.at[idx_v.at[pl.ds(q * 128, 128)]],
                    rows_v.at[pl.ds(q * 128, 128)],
                    sem,
                )
                for q in range(ni // 128)
            ]
            for c in copies:
                c.wait()

            def row_body(r, c2):
                wvec = w_v[pl.ds(r * 3, 16)]
                for j in range(feat // 16):
                    sl = pl.ds(j * 16, 16)
                    acc = f1_v[r, sl]
                    for k in range(3):
                        acc = acc + wvec[k] * rows_v[r * 3 + k, sl]
                    f1_v[r, sl] = acc
                return c2

            lax.fori_loop(0, _RPC, row_body, 0, unroll=2)
            pltpu.sync_copy(f1_v, out_hbm.at[pl.ds(row0, _RPC)])
            return carry

        lax.fori_loop(0, nchunks, chunk_body, 0)

    return sc_combine


def kernel(feats1, points1, feats2, points2, W1, b1, g1, beta1, W2, b2, g2,
           beta2):
    B, N1, C1 = feats1.shape
    _, N2, C2 = feats2.shape
    C = W1.shape[0]
    x1 = feats1.reshape(B * N1, C1)
    x2 = feats2.reshape(B * N2, C2)
    b1r = b1.reshape(1, C)
    g1r = g1.reshape(1, C)
    beta1r = beta1.reshape(1, C)
    b2r = b2.reshape(1, C)
    g2r = g2.reshape(1, C)
    beta2r = beta2.reshape(1, C)

    chunk = 4096
    nchunks = (B * N1) // chunk
    stats = pl.pallas_call(
        _stats1_kernel,
        grid=(nchunks,),
        in_specs=[
            pl.BlockSpec((chunk, C1), lambda i: (i, 0)),
            pl.BlockSpec((C, C1), lambda i: (0, 0)),
            pl.BlockSpec((1, C), lambda i: (0, 0)),
        ],
        out_specs=pl.BlockSpec((2, C), lambda i: (0, 0)),
        out_shape=jax.ShapeDtypeStruct((2, C), jnp.float32),
    )(x1, W1, b1r)

    f2 = pl.pallas_call(
        _f2_kernel,
        out_shape=jax.ShapeDtypeStruct((B * N2, 2 * C), jnp.float32),
    )(x2, W2, b2r, g2r, beta2r)

    p2t = points2.transpose(0, 2, 1)

    nba = _SPLIT
    nbb = N1 // _M - nba
    n1a = nba * _M

    f1a, w3, idx3 = pl.pallas_call(
        functools.partial(_knn_emit_kernel, n_rows=float(B * N1), n2=N2),
        grid=(B, nba),
        in_specs=[
            pl.BlockSpec((1, _M, C1), lambda b, n: (b, n, 0)),
            pl.BlockSpec((1, _M, 3), lambda b, n: (b, n, 0)),
            pl.BlockSpec((1, 3, N2), lambda b, n: (b, 0, 0)),
            pl.BlockSpec((C, C1), lambda b, n: (0, 0)),
            pl.BlockSpec((1, C), lambda b, n: (0, 0)),
            pl.BlockSpec((1, C), lambda b, n: (0, 0)),
            pl.BlockSpec((1, C), lambda b, n: (0, 0)),
            pl.BlockSpec((2, C), lambda b, n: (0, 0)),
        ],
        out_specs=(
            pl.BlockSpec((1, _M, C), lambda b, n: (b, n, 0)),
            pl.BlockSpec((1, _M, 3), lambda b, n: (b, n, 0)),
            pl.BlockSpec((1, _M, 3), lambda b, n: (b, n, 0)),
        ),
        out_shape=(
            jax.ShapeDtypeStruct((B, n1a, C), jnp.float32),
            jax.ShapeDtypeStruct((B, n1a, 3), jnp.float32),
            jax.ShapeDtypeStruct((B, n1a, 3), jnp.int32),
        ),
    )(feats1, points1, p2t, W1, b1r, g1r, beta1r, stats)

    sc_combine = _make_sc_combine(B * n1a, C)
    out_a = sc_combine(
        f2,
        idx3.reshape(B * n1a * 3),
        w3.reshape(B * n1a * 3),
        f1a.reshape(B * n1a, C),
    )

    out_b = pl.pallas_call(
        functools.partial(_knn_onehot_kernel, n_rows=float(B * N1), n2=N2,
                          feat=C),
        grid=(B, nbb),
        in_specs=[
            pl.BlockSpec((1, _M, C1), lambda b, n: (b, n + nba, 0)),
            pl.BlockSpec((1, _M, 3), lambda b, n: (b, n + nba, 0)),
            pl.BlockSpec((1, 3, N2), lambda b, n: (b, 0, 0)),
            pl.BlockSpec((1, N2, 2 * C), lambda b, n: (b, 0, 0)),
            pl.BlockSpec((C, C1), lambda b, n: (0, 0)),
            pl.BlockSpec((1, C), lambda b, n: (0, 0)),
            pl.BlockSpec((1, C), lambda b, n: (0, 0)),
            pl.BlockSpec((1, C), lambda b, n: (0, 0)),
            pl.BlockSpec((2, C), lambda b, n: (0, 0)),
        ],
        out_specs=pl.BlockSpec((1, _M, C), lambda b, n: (b, n, 0)),
        out_shape=jax.ShapeDtypeStruct((B, N1 - n1a, C), jnp.float32),
    )(feats1, points1, p2t, f2.reshape(B, N2, 2 * C), W1, b1r, g1r, beta1r,
      stats)

    out = jnp.concatenate([out_a.reshape(B, n1a, C), out_b], axis=1)
    return (out, points1)


# speedup vs baseline: 1.0634x; 1.0203x over previous
import functools

import jax
import jax.numpy as jnp
from jax import lax
from jax.experimental import pallas as pl
from jax.experimental.pallas import tpu as pltpu
from jax.experimental.pallas import tpu_sc as plsc

_EPS = 1e-08
_M = 1024
_RPC = 128
_SPLIT = 5


def _stats1_kernel(x_ref, w_ref, b_ref, acc_ref):
    i = pl.program_id(0)
    y = jnp.dot(x_ref[...], w_ref[...].T, preferred_element_type=jnp.float32)
    y = y + b_ref[...]
    s = jnp.sum(y, axis=0, keepdims=True)
    ss = jnp.sum(y * y, axis=0, keepdims=True)
    blk = jnp.concatenate([s, ss], axis=0)

    @pl.when(i == 0)
    def _():
        acc_ref[...] = blk

    @pl.when(i > 0)
    def _():
        acc_ref[...] += blk


def _f2_kernel(x_ref, w_ref, b_ref, g_ref, beta_ref, out_ref):
    y = jnp.dot(x_ref[...], w_ref[...].T, preferred_element_type=jnp.float32)
    y = y + b_ref[...]
    m = jnp.mean(y, axis=0, keepdims=True)
    v = jnp.mean((y - m) ** 2, axis=0, keepdims=True)
    f2 = jnp.maximum(
        (y - m) / jnp.sqrt(v + 1e-5) * g_ref[...] + beta_ref[...], 0.0
    )
    out_ref[...] = jnp.concatenate([f2, jnp.zeros_like(f2)], axis=1)


def _f1_and_top3(x1_ref, p1_ref, p2t_ref, w1_ref, b1_ref, g1_ref, beta1_ref,
                 stats_ref, n_rows):
    mean = stats_ref[0:1, :] / n_rows
    var = stats_ref[1:2, :] / n_rows - mean * mean
    x1 = x1_ref[0]
    y1 = jnp.dot(x1, w1_ref[...].T, preferred_element_type=jnp.float32)
    y1 = y1 + b1_ref[...]
    f1 = jnp.maximum(
        (y1 - mean) / jnp.sqrt(var + 1e-5) * g1_ref[...] + beta1_ref[...], 0.0
    )

    p1 = p1_ref[0]
    p2t = p2t_ref[0]
    q2 = jnp.sum(p1 * p1, axis=1, keepdims=True)
    r2 = jnp.sum(p2t * p2t, axis=0, keepdims=True)
    cross = jnp.dot(p1, p2t, preferred_element_type=jnp.float32)
    d2 = jnp.maximum(q2 + r2 - 2.0 * cross, 0.0)

    iota = jax.lax.broadcasted_iota(jnp.int32, d2.shape, 1)
    hi = jnp.int32(-2048)
    ukey = (jax.lax.bitcast_convert_type(d2, jnp.int32) & hi) | iota
    fkey = jax.lax.bitcast_convert_type(ukey + jnp.int32(0x00800000),
                                        jnp.float32)
    fbig = jnp.float32(3e38)
    m1 = jnp.min(fkey, axis=1, keepdims=True)
    u2 = jnp.where(fkey == m1, fbig, fkey)
    m2 = jnp.min(u2, axis=1, keepdims=True)
    m3 = jnp.min(jnp.where(u2 == m2, fbig, u2), axis=1, keepdims=True)
    ikeys = [jax.lax.bitcast_convert_type(mk, jnp.int32)
             for mk in (m1, m2, m3)]
    mins = [
        jax.lax.bitcast_convert_type(
            (ik - jnp.int32(0x00800000)) & hi, jnp.float32)
        for ik in ikeys
    ]
    recips = [1.0 / (jnp.sqrt(mn) + _EPS) for mn in mins]
    norm = recips[0] + recips[1] + recips[2]
    weights = [rc / norm for rc in recips]
    return f1, fkey, (m1, m2, m3), ikeys, weights


def _knn_emit_kernel(x1_ref, p1_ref, p2t_ref, w1_ref, b1_ref, g1_ref,
                     beta1_ref, stats_ref, f1_ref, w3_ref, idx3_ref,
                     *, n_rows, n2):
    f1, _, _, ikeys, weights = _f1_and_top3(
        x1_ref, p1_ref, p2t_ref, w1_ref, b1_ref, g1_ref, beta1_ref,
        stats_ref, n_rows)
    f1_ref[0] = f1
    cols = [ik & jnp.int32(2047) for ik in ikeys]
    gbase = pl.program_id(0) * n2
    w3_ref[0] = jnp.concatenate(weights, axis=1)
    idx3_ref[0] = jnp.concatenate(cols, axis=1) + gbase


def _knn_onehot_kernel(x1_ref, p1_ref, p2t_ref, f2_ref, w1_ref, b1_ref,
                       g1_ref, beta1_ref, stats_ref, out_ref,
                       *, n_rows, n2, feat):
    f1, fkey, (m1, m2, m3), _, weights = _f1_and_top3(
        x1_ref, p1_ref, p2t_ref, w1_ref, b1_ref, g1_ref, beta1_ref,
        stats_ref, n_rows)
    oh = jnp.where(
        fkey == m1, weights[0],
        jnp.where(fkey == m2, weights[1],
                  jnp.where(fkey == m3, weights[2], 0.0)),
    )
    f2b = f2_ref[0][:, :feat]
    new = jnp.dot(oh, f2b, preferred_element_type=jnp.float32)
    out_ref[0] = f1 + new


def _make_sc_combine(n_rows, feat):
    info = plsc.get_sparse_core_info()
    nw = info.num_cores * info.num_subcores
    rows_per_w = n_rows // nw
    nchunks = rows_per_w // _RPC
    ni = _RPC * 3
    mesh = plsc.VectorSubcoreMesh(core_axis_name="c", subcore_axis_name="s")

    @functools.partial(
        pl.kernel, mesh=mesh,
        out_type=jax.ShapeDtypeStruct((n_rows, feat), jnp.float32),
        scratch_types=[
            pltpu.VMEM((ni,), jnp.int32),
            pltpu.VMEM((ni + 16,), jnp.float32),
            pltpu.VMEM((ni, 2 * feat), jnp.float32),
            pltpu.VMEM((_RPC, feat), jnp.float32),
            pltpu.SemaphoreType.DMA,
        ],
    )
    def sc_combine(f2_hbm, idx_hbm, w_hbm, f1_hbm, out_hbm,
                   idx_v, w_v, rows_v, f1_v, sem):
        wid = lax.axis_index("s") * info.num_cores + lax.axis_index("c")
        base = wid * rows_per_w

        def chunk_body(g, carry):
            row0 = base + g * _RPC
            pltpu.sync_copy(idx_hbm.at[pl.ds(row0 * 3, ni)], idx_v)
            pltpu.sync_copy(w_hbm.at[pl.ds(row0 * 3, ni)],
                            w_v.at[pl.ds(0, ni)])
            pltpu.sync_copy(f1_hbm.at[pl.ds(row0, _RPC)], f1_v)
            copies = [
                pltpu.async_copy(
                    f2_hbm.at[idx_v.at[pl.ds(q * 128, 128)]],
                    rows_v.at[pl.ds(q * 128, 128)],
                    sem,
                )
                for q in range(ni // 128)
            ]
            for c in copies:
                c.wait()

            def row_body(r, c2):
                wvec = w_v[pl.ds(r * 3, 16)]
                for j in range(feat // 16):
                    sl = pl.ds(j * 16, 16)
                    acc = f1_v[r, sl]
                    for k in range(3):
                        acc = acc + wvec[k] * rows_v[r * 3 + k, sl]
                    f1_v[r, sl] = acc
                return c2

            lax.fori_loop(0, _RPC, row_body, 0, unroll=2)
            pltpu.sync_copy(f1_v, out_hbm.at[pl.ds(row0, _RPC)])
            return carry

        lax.fori_loop(0, nchunks, chunk_body, 0)

    return sc_combine


def kernel(feats1, points1, feats2, points2, W1, b1, g1, beta1, W2, b2, g2,
           beta2):
    B, N1, C1 = feats1.shape
    _, N2, C2 = feats2.shape
    C = W1.shape[0]
    x1 = feats1.reshape(B * N1, C1)
    x2 = feats2.reshape(B * N2, C2)
    b1r = b1.reshape(1, C)
    g1r = g1.reshape(1, C)
    beta1r = beta1.reshape(1, C)
    b2r = b2.reshape(1, C)
    g2r = g2.reshape(1, C)
    beta2r = beta2.reshape(1, C)

    chunk = 4096
    nchunks = (B * N1) // chunk
    stats = pl.pallas_call(
        _stats1_kernel,
        grid=(nchunks,),
        in_specs=[
            pl.BlockSpec((chunk, C1), lambda i: (i, 0)),
            pl.BlockSpec((C, C1), lambda i: (0, 0)),
            pl.BlockSpec((1, C), lambda i: (0, 0)),
        ],
        out_specs=pl.BlockSpec((2, C), lambda i: (0, 0)),
        out_shape=jax.ShapeDtypeStruct((2, C), jnp.float32),
    )(x1, W1, b1r)

    f2 = pl.pallas_call(
        _f2_kernel,
        out_shape=jax.ShapeDtypeStruct((B * N2, 2 * C), jnp.float32),
    )(x2, W2, b2r, g2r, beta2r)

    p2t = points2.transpose(0, 2, 1)

    nba = _SPLIT
    nbb = N1 // _M - nba
    n1a = nba * _M

    f1a, w3, idx3 = pl.pallas_call(
        functools.partial(_knn_emit_kernel, n_rows=float(B * N1), n2=N2),
        grid=(B, nba),
        in_specs=[
            pl.BlockSpec((1, _M, C1), lambda b, n: (b, n, 0)),
            pl.BlockSpec((1, _M, 3), lambda b, n: (b, n, 0)),
            pl.BlockSpec((1, 3, N2), lambda b, n: (b, 0, 0)),
            pl.BlockSpec((C, C1), lambda b, n: (0, 0)),
            pl.BlockSpec((1, C), lambda b, n: (0, 0)),
            pl.BlockSpec((1, C), lambda b, n: (0, 0)),
            pl.BlockSpec((1, C), lambda b, n: (0, 0)),
            pl.BlockSpec((2, C), lambda b, n: (0, 0)),
        ],
        out_specs=(
            pl.BlockSpec((1, _M, C), lambda b, n: (b, n, 0)),
            pl.BlockSpec((1, _M, 3), lambda b, n: (b, n, 0)),
            pl.BlockSpec((1, _M, 3), lambda b, n: (b, n, 0)),
        ),
        out_shape=(
            jax.ShapeDtypeStruct((B, n1a, C), jnp.float32),
            jax.ShapeDtypeStruct((B, n1a, 3), jnp.float32),
            jax.ShapeDtypeStruct((B, n1a, 3), jnp.int32),
        ),
    )(feats1, points1, p2t, W1, b1r, g1r, beta1r, stats)

    sc_combine = _make_sc_combine(B * n1a, C)
    out_a = sc_combine(
        f2,
        idx3.reshape(B * n1a * 3),
        w3.reshape(B * n1a * 3),
        f1a.reshape(B * n1a, C),
    )

    out_b = pl.pallas_call(
        functools.partial(_knn_onehot_kernel, n_rows=float(B * N1), n2=N2,
                          feat=C),
        grid=(B, nbb),
        in_specs=[
            pl.BlockSpec((1, _M, C1), lambda b, n: (b, n + nba, 0)),
            pl.BlockSpec((1, _M, 3), lambda b, n: (b, n + nba, 0)),
            pl.BlockSpec((1, 3, N2), lambda b, n: (b, 0, 0)),
            pl.BlockSpec((1, N2, 2 * C), lambda b, n: (b, 0, 0)),
            pl.BlockSpec((C, C1), lambda b, n: (0, 0)),
            pl.BlockSpec((1, C), lambda b, n: (0, 0)),
            pl.BlockSpec((1, C), lambda b, n: (0, 0)),
            pl.BlockSpec((1, C), lambda b, n: (0, 0)),
            pl.BlockSpec((2, C), lambda b, n: (0, 0)),
        ],
        out_specs=pl.BlockSpec((1, _M, C), lambda b, n: (b, n, 0)),
        out_shape=jax.ShapeDtypeStruct((B, N1 - n1a, C), jnp.float32),
    )(feats1, points1, p2t, f2.reshape(B, N2, 2 * C), W1, b1r, g1r, beta1r,
      stats)

    out = jnp.concatenate([out_a.reshape(B, n1a, C), out_b], axis=1)
    return (out, points1)
